# baseline (speedup 1.0000x reference)
import jax
import jax.numpy as jnp
from jax import lax
from jax.experimental import pallas as pl
from jax.experimental.pallas import tpu as pltpu
from jax.experimental.pallas import tpu_sc as plsc

N = 320000
D = 128
S = 10000
SPAD = 10240
SEG_PER_TILE = SPAD // 16
BL = 128
BH = 64
NT = 16
NW = 32
G1 = N // BL
GPT = G1 // NT
EXTRA_T = G1 - GPT * NT
PAIRS1 = GPT // 2
G3 = N // BH
GPW = G3 // NW
EXTRA_W = G3 - GPW * NW
PAIRS3 = GPW // 2

_mesh = plsc.VectorSubcoreMesh(core_axis_name="c", subcore_axis_name="s")


def _body(x_hbm, idx_hbm, gam_hbm, bet_hbm, out_hbm,
          ssum, scnt, xa, xb, ia, ib, ica, icb, i3a, i3b, onesv, gv, bv,
          sia, sib, sxa, sxb, soa, sob, sga, sgb, sca, scb, ssa, ssb):
    c = lax.axis_index("c")
    s = lax.axis_index("s")
    wid = s * 2 + c
    seg0 = s * SEG_PER_TILE

    def zrow(r, _):
        for j in range(D // 16):
            xa[r, pl.ds(16 * j, 16)] = jnp.zeros((16,), jnp.float32)
        return 0

    lax.fori_loop(0, BL, zrow, 0)

    def zv(k, _):
        onesv[pl.ds(16 * k, 16)] = jnp.zeros((16,), jnp.float32)
        return 0

    lax.fori_loop(0, BL // 16, zv, 0)

    for g in range(SEG_PER_TILE // BL):
        seg = seg0 + g * BL
        pltpu.async_copy(xa, ssum.at[pl.ds(seg, BL)], soa)
        pltpu.async_copy(onesv, scnt.at[pl.ds(seg, BL)], sob)
    for g in range(SEG_PER_TILE // BL):
        seg = seg0 + g * BL
        pltpu.make_async_copy(xa, ssum.at[pl.ds(seg, BL)], soa).wait()
        pltpu.make_async_copy(onesv, scnt.at[pl.ds(seg, BL)], sob).wait()

    def ov(k, _):
        onesv[pl.ds(16 * k, 16)] = jnp.ones((16,), jnp.float32)
        return 0

    lax.fori_loop(0, BL // 16, ov, 0)
    pltpu.sync_copy(gam_hbm, gv)
    pltpu.sync_copy(bet_hbm, bv)
    plsc.subcore_barrier()

    def row1(i):
        return (s * GPT + i) * BL

    def icopy(src, dst):
        for k in range(BL // 16):
            dst[pl.ds(16 * k, 16)] = src[pl.ds(16 * k, 16)]

    pltpu.async_copy(idx_hbm.at[pl.ds(row1(0), BL)], ia, sia)
    pltpu.async_copy(x_hbm.at[pl.ds(row1(0), BL)], xa, sxa)

    def acc(t, _):
        i0 = 2 * t
        i1 = i0 + 1

        @pl.when(t > 0)
        def _():
            pltpu.make_async_copy(xb, ssum.at[ib], ssb).wait()

        pltpu.async_copy(idx_hbm.at[pl.ds(row1(i1), BL)], ib, sib)
        pltpu.async_copy(x_hbm.at[pl.ds(row1(i1), BL)], xb, sxb)
        pltpu.make_async_copy(idx_hbm.at[pl.ds(row1(i0), BL)], ia, sia).wait()
        pltpu.make_async_copy(x_hbm.at[pl.ds(row1(i0), BL)], xa, sxa).wait()
        pltpu.async_copy(xa, ssum.at[ia], ssa, add=True)

        @pl.when(t > 0)
        def _():
            pltpu.make_async_copy(onesv, scnt.at[ica], sca).wait()

        icopy(ia, ica)
        pltpu.async_copy(onesv, scnt.at[ica], sca, add=True)

        @pl.when(t + 1 < PAIRS1)
        def _():
            pltpu.make_async_copy(xa, ssum.at[ia], ssa).wait()
            pltpu.async_copy(idx_hbm.at[pl.ds(row1(i0 + 2), BL)], ia, sia)
            pltpu.async_copy(x_hbm.at[pl.ds(row1(i0 + 2), BL)], xa, sxa)

        pltpu.make_async_copy(idx_hbm.at[pl.ds(row1(i1), BL)], ib, sib).wait()
        pltpu.make_async_copy(x_hbm.at[pl.ds(row1(i1), BL)], xb, sxb).wait()
        pltpu.async_copy(xb, ssum.at[ib], ssb, add=True)

        @pl.when(t > 0)
        def _():
            pltpu.make_async_copy(onesv, scnt.at[icb], scb).wait()

        icopy(ib, icb)
        pltpu.async_copy(onesv, scnt.at[icb], scb, add=True)
        return 0

    lax.fori_loop(0, PAIRS1, acc, 0)
    pltpu.make_async_copy(onesv, scnt.at[ica], sca).wait()
    pltpu.make_async_copy(onesv, scnt.at[icb], scb).wait()

    pltpu.make_async_copy(xa, ssum.at[ia], ssa).wait()
    pltpu.make_async_copy(xb, ssum.at[ib], ssb).wait()

    @pl.when(s < EXTRA_T)
    def _():
        row = (NT * GPT + s) * BL
        pltpu.sync_copy(idx_hbm.at[pl.ds(row, BL)], ia)
        pltpu.sync_copy(x_hbm.at[pl.ds(row, BL)], xa)
        pltpu.sync_copy(xa, ssum.at[ia], add=True)
        pltpu.sync_copy(onesv, scnt.at[ia], add=True)

    plsc.subcore_barrier()

    def fin(g, _):
        seg = seg0 + g * BL
        pltpu.sync_copy(ssum.at[pl.ds(seg, BL)], xa)
        pltpu.sync_copy(scnt.at[pl.ds(seg, BL)], onesv)
        for k in range(BL // 16):
            inv = 1.0 / jnp.maximum(onesv[pl.ds(16 * k, 16)], 1.0)
            for r in range(16):
                rr = 16 * k + r
                invr = jnp.broadcast_to(lax.slice(inv, (r,), (r + 1,)), (16,))
                for j in range(D // 16):
                    sl = pl.ds(16 * j, 16)
                    xa[rr, sl] = xa[rr, sl] * invr
        pltpu.sync_copy(xa, ssum.at[pl.ds(seg, BL)])
        return 0

    lax.fori_loop(0, SEG_PER_TILE // BL, fin, 0)
    plsc.subcore_barrier()

    gs = [gv[pl.ds(16 * j, 16)] for j in range(D // 16)]
    cs = [gs[j] + bv[pl.ds(16 * j, 16)] for j in range(D // 16)]

    def row3(i):
        return (wid * GPW + i) * BH

    def affine(buf):
        def rows(r, _):
            for j in range(D // 16):
                sl = pl.ds(16 * j, 16)
                buf[BH + r, sl] = (buf[r, sl] - buf[BH + r, sl]) * gs[j] + cs[j]
            return 0

        lax.fori_loop(0, BH, rows, 0)

    pltpu.async_copy(idx_hbm.at[pl.ds(row3(0), BH)], i3a, sia)
    pltpu.async_copy(x_hbm.at[pl.ds(row3(0), BH)], xa.at[pl.ds(0, BH)], sxa)
    pltpu.async_copy(idx_hbm.at[pl.ds(row3(1), BH)], i3b, sib)
    pltpu.async_copy(x_hbm.at[pl.ds(row3(1), BH)], xb.at[pl.ds(0, BH)], sxb)
    pltpu.make_async_copy(idx_hbm.at[pl.ds(row3(0), BH)], i3a, sia).wait()
    pltpu.async_copy(ssum.at[i3a], xa.at[pl.ds(BH, BH)], sga)

    def emit(t, _):
        i0 = 2 * t
        i1 = i0 + 1

        pltpu.make_async_copy(idx_hbm.at[pl.ds(row3(i1), BH)], i3b, sib).wait()

        @pl.when(t > 0)
        def _():
            pltpu.make_async_copy(xb.at[pl.ds(BH, BH)],
                                  out_hbm.at[pl.ds(row3(i1 - 2), BH)], sob).wait()

        pltpu.async_copy(ssum.at[i3b], xb.at[pl.ds(BH, BH)], sgb)

        pltpu.make_async_copy(x_hbm.at[pl.ds(row3(i0), BH)],
                              xa.at[pl.ds(0, BH)], sxa).wait()
        pltpu.make_async_copy(ssum.at[i3a], xa.at[pl.ds(BH, BH)], sga).wait()
        affine(xa)
        pltpu.async_copy(xa.at[pl.ds(BH, BH)], out_hbm.at[pl.ds(row3(i0), BH)], soa)

        @pl.when(t + 1 < PAIRS3)
        def _():
            pltpu.async_copy(idx_hbm.at[pl.ds(row3(i0 + 2), BH)], i3a, sia)
            pltpu.async_copy(x_hbm.at[pl.ds(row3(i0 + 2), BH)],
                             xa.at[pl.ds(0, BH)], sxa)

        pltpu.make_async_copy(x_hbm.at[pl.ds(row3(i1), BH)],
                              xb.at[pl.ds(0, BH)], sxb).wait()
        pltpu.make_async_copy(ssum.at[i3b], xb.at[pl.ds(BH, BH)], sgb).wait()
        affine(xb)
        pltpu.async_copy(xb.at[pl.ds(BH, BH)], out_hbm.at[pl.ds(row3(i1), BH)], sob)

        @pl.when(t + 1 < PAIRS3)
        def _():
            pltpu.async_copy(idx_hbm.at[pl.ds(row3(i1 + 2), BH)], i3b, sib)
            pltpu.async_copy(x_hbm.at[pl.ds(row3(i1 + 2), BH)],
                             xb.at[pl.ds(0, BH)], sxb)
            pltpu.make_async_copy(idx_hbm.at[pl.ds(row3(i0 + 2), BH)], i3a, sia).wait()
            pltpu.make_async_copy(xa.at[pl.ds(BH, BH)],
                                  out_hbm.at[pl.ds(row3(i0), BH)], soa).wait()
            pltpu.async_copy(ssum.at[i3a], xa.at[pl.ds(BH, BH)], sga)

        return 0

    lax.fori_loop(0, PAIRS3, emit, 0)
    pltpu.make_async_copy(xa.at[pl.ds(BH, BH)],
                          out_hbm.at[pl.ds(row3(0), BH)], soa).wait()
    pltpu.make_async_copy(xb.at[pl.ds(BH, BH)],
                          out_hbm.at[pl.ds(row3(1), BH)], sob).wait()

    @pl.when(wid < EXTRA_W)
    def _():
        row = (NW * GPW + wid) * BH
        pltpu.sync_copy(idx_hbm.at[pl.ds(row, BH)], i3a)
        pltpu.sync_copy(x_hbm.at[pl.ds(row, BH)], xa.at[pl.ds(0, BH)])
        pltpu.sync_copy(ssum.at[i3a], xa.at[pl.ds(BH, BH)])
        affine(xa)
        pltpu.sync_copy(xa.at[pl.ds(BH, BH)], out_hbm.at[pl.ds(row, BH)])


_sc_kernel = pl.kernel(
    _body,
    out_type=jax.ShapeDtypeStruct((N, D), jnp.float32),
    mesh=_mesh,
    scratch_types=[
        pltpu.VMEM_SHARED((SPAD, D), jnp.float32),
        pltpu.VMEM_SHARED((SPAD,), jnp.float32),
        pltpu.VMEM((BL, D), jnp.float32),
        pltpu.VMEM((BL, D), jnp.float32),
        pltpu.VMEM((BL,), jnp.int32),
        pltpu.VMEM((BL,), jnp.int32),
        pltpu.VMEM((BL,), jnp.int32),
        pltpu.VMEM((BL,), jnp.int32),
        pltpu.VMEM((BH,), jnp.int32),
        pltpu.VMEM((BH,), jnp.int32),
        pltpu.VMEM((BL,), jnp.float32),
        pltpu.VMEM((D,), jnp.float32),
        pltpu.VMEM((D,), jnp.float32),
        pltpu.SemaphoreType.DMA,
        pltpu.SemaphoreType.DMA,
        pltpu.SemaphoreType.DMA,
        pltpu.SemaphoreType.DMA,
        pltpu.SemaphoreType.DMA,
        pltpu.SemaphoreType.DMA,
        pltpu.SemaphoreType.DMA,
        pltpu.SemaphoreType.DMA,
        pltpu.SemaphoreType.DMA,
        pltpu.SemaphoreType.DMA,
        pltpu.SemaphoreType.DMA,
        pltpu.SemaphoreType.DMA,
    ],
)


@jax.jit
def kernel(x, index, gamma, beta):
    idx = index.astype(jnp.int32)
    return _sc_kernel(x, idx,
                      gamma.reshape(D).astype(jnp.float32),
                      beta.reshape(D).astype(jnp.float32))


# speedup vs baseline: 1.0129x; 1.0129x over previous
import jax
import jax.numpy as jnp
from jax import lax
from jax.experimental import pallas as pl
from jax.experimental.pallas import tpu as pltpu
from jax.experimental.pallas import tpu_sc as plsc

N = 320000
D = 128
S = 10000
SPAD = 10240
SEG_PER_TILE = SPAD // 16
BL = 128
BH = 64
NT = 16
NW = 32
G1 = N // BL
GPT = G1 // NT
EXTRA_T = G1 - GPT * NT
PAIRS1 = GPT // 2
G3 = N // BH
GPW = G3 // NW
EXTRA_W = G3 - GPW * NW
PAIRS3 = GPW // 2

_mesh = plsc.VectorSubcoreMesh(core_axis_name="c", subcore_axis_name="s")


def _body(x_hbm, idx_hbm, gam_hbm, bet_hbm, out_hbm,
          ssum, scnt, xa, xb, ia, ib, ica, icb, i3a, i3b, onesv, gv, bv,
          sia, sib, sxa, sxb, soa, sob, sga, sgb, sca, scb):
    c = lax.axis_index("c")
    s = lax.axis_index("s")
    wid = s * 2 + c
    seg0 = s * SEG_PER_TILE

    def zrow(r, _):
        for j in range(D // 16):
            xa[r, pl.ds(16 * j, 16)] = jnp.zeros((16,), jnp.float32)
        return 0

    lax.fori_loop(0, BL, zrow, 0)

    def zv(k, _):
        onesv[pl.ds(16 * k, 16)] = jnp.zeros((16,), jnp.float32)
        return 0

    lax.fori_loop(0, BL // 16, zv, 0)

    for g in range(SEG_PER_TILE // BL):
        seg = seg0 + g * BL
        pltpu.async_copy(xa, ssum.at[pl.ds(seg, BL)], soa)
        pltpu.async_copy(onesv, scnt.at[pl.ds(seg, BL)], sob)
    for g in range(SEG_PER_TILE // BL):
        seg = seg0 + g * BL
        pltpu.make_async_copy(xa, ssum.at[pl.ds(seg, BL)], soa).wait()
        pltpu.make_async_copy(onesv, scnt.at[pl.ds(seg, BL)], sob).wait()

    def ov(k, _):
        onesv[pl.ds(16 * k, 16)] = jnp.ones((16,), jnp.float32)
        return 0

    lax.fori_loop(0, BL // 16, ov, 0)
    pltpu.sync_copy(gam_hbm, gv)
    pltpu.sync_copy(bet_hbm, bv)
    plsc.subcore_barrier()

    def row1(i):
        return (s * GPT + i) * BL

    def icopy(src, dst):
        for k in range(BL // 16):
            dst[pl.ds(16 * k, 16)] = src[pl.ds(16 * k, 16)]

    pltpu.async_copy(idx_hbm.at[pl.ds(row1(0), BL)], ia, sia)
    pltpu.async_copy(x_hbm.at[pl.ds(row1(0), BL)], xa, sxa)

    def acc(t, _):
        i0 = 2 * t
        i1 = i0 + 1
        pltpu.async_copy(idx_hbm.at[pl.ds(row1(i1), BL)], ib, sib)
        pltpu.async_copy(x_hbm.at[pl.ds(row1(i1), BL)], xb, sxb)
        pltpu.make_async_copy(idx_hbm.at[pl.ds(row1(i0), BL)], ia, sia).wait()
        pltpu.make_async_copy(x_hbm.at[pl.ds(row1(i0), BL)], xa, sxa).wait()
        pltpu.sync_copy(xa, ssum.at[ia], add=True)

        @pl.when(t > 0)
        def _():
            pltpu.make_async_copy(onesv, scnt.at[ica], sca).wait()

        icopy(ia, ica)
        pltpu.async_copy(onesv, scnt.at[ica], sca, add=True)

        @pl.when(t + 1 < PAIRS1)
        def _():
            pltpu.async_copy(idx_hbm.at[pl.ds(row1(i0 + 2), BL)], ia, sia)
            pltpu.async_copy(x_hbm.at[pl.ds(row1(i0 + 2), BL)], xa, sxa)

        pltpu.make_async_copy(idx_hbm.at[pl.ds(row1(i1), BL)], ib, sib).wait()
        pltpu.make_async_copy(x_hbm.at[pl.ds(row1(i1), BL)], xb, sxb).wait()
        pltpu.sync_copy(xb, ssum.at[ib], add=True)

        @pl.when(t > 0)
        def _():
            pltpu.make_async_copy(onesv, scnt.at[icb], scb).wait()

        icopy(ib, icb)
        pltpu.async_copy(onesv, scnt.at[icb], scb, add=True)
        return 0

    lax.fori_loop(0, PAIRS1, acc, 0)
    pltpu.make_async_copy(onesv, scnt.at[ica], sca).wait()
    pltpu.make_async_copy(onesv, scnt.at[icb], scb).wait()

    @pl.when(s < EXTRA_T)
    def _():
        row = (NT * GPT + s) * BL
        pltpu.sync_copy(idx_hbm.at[pl.ds(row, BL)], ia)
        pltpu.sync_copy(x_hbm.at[pl.ds(row, BL)], xa)
        pltpu.sync_copy(xa, ssum.at[ia], add=True)
        pltpu.sync_copy(onesv, scnt.at[ia], add=True)

    plsc.subcore_barrier()

    def fin(g, _):
        seg = seg0 + g * BL
        pltpu.sync_copy(ssum.at[pl.ds(seg, BL)], xa)
        pltpu.sync_copy(scnt.at[pl.ds(seg, BL)], onesv)
        for k in range(BL // 16):
            inv = 1.0 / jnp.maximum(onesv[pl.ds(16 * k, 16)], 1.0)
            for r in range(16):
                rr = 16 * k + r
                invr = jnp.broadcast_to(lax.slice(inv, (r,), (r + 1,)), (16,))
                for j in range(D // 16):
                    sl = pl.ds(16 * j, 16)
                    xa[rr, sl] = xa[rr, sl] * invr
        pltpu.sync_copy(xa, ssum.at[pl.ds(seg, BL)])
        return 0

    lax.fori_loop(0, SEG_PER_TILE // BL, fin, 0)
    plsc.subcore_barrier()

    gs = [gv[pl.ds(16 * j, 16)] for j in range(D // 16)]
    cs = [gs[j] + bv[pl.ds(16 * j, 16)] for j in range(D // 16)]

    def row3(i):
        return (wid * GPW + i) * BH

    def affine(buf):
        def rows(r, _):
            for j in range(D // 16):
                sl = pl.ds(16 * j, 16)
                buf[BH + r, sl] = (buf[r, sl] - buf[BH + r, sl]) * gs[j] + cs[j]
            return 0

        lax.fori_loop(0, BH, rows, 0)

    pltpu.async_copy(idx_hbm.at[pl.ds(row3(0), BH)], i3a, sia)
    pltpu.async_copy(x_hbm.at[pl.ds(row3(0), BH)], xa.at[pl.ds(0, BH)], sxa)
    pltpu.async_copy(idx_hbm.at[pl.ds(row3(1), BH)], i3b, sib)
    pltpu.async_copy(x_hbm.at[pl.ds(row3(1), BH)], xb.at[pl.ds(0, BH)], sxb)
    pltpu.make_async_copy(idx_hbm.at[pl.ds(row3(0), BH)], i3a, sia).wait()
    pltpu.async_copy(ssum.at[i3a], xa.at[pl.ds(BH, BH)], sga)

    def emit(t, _):
        i0 = 2 * t
        i1 = i0 + 1

        pltpu.make_async_copy(idx_hbm.at[pl.ds(row3(i1), BH)], i3b, sib).wait()

        @pl.when(t > 0)
        def _():
            pltpu.make_async_copy(xb.at[pl.ds(BH, BH)],
                                  out_hbm.at[pl.ds(row3(i1 - 2), BH)], sob).wait()

        pltpu.async_copy(ssum.at[i3b], xb.at[pl.ds(BH, BH)], sgb)

        pltpu.make_async_copy(x_hbm.at[pl.ds(row3(i0), BH)],
                              xa.at[pl.ds(0, BH)], sxa).wait()
        pltpu.make_async_copy(ssum.at[i3a], xa.at[pl.ds(BH, BH)], sga).wait()
        affine(xa)
        pltpu.async_copy(xa.at[pl.ds(BH, BH)], out_hbm.at[pl.ds(row3(i0), BH)], soa)

        @pl.when(t + 1 < PAIRS3)
        def _():
            pltpu.async_copy(idx_hbm.at[pl.ds(row3(i0 + 2), BH)], i3a, sia)
            pltpu.async_copy(x_hbm.at[pl.ds(row3(i0 + 2), BH)],
                             xa.at[pl.ds(0, BH)], sxa)

        pltpu.make_async_copy(x_hbm.at[pl.ds(row3(i1), BH)],
                              xb.at[pl.ds(0, BH)], sxb).wait()
        pltpu.make_async_copy(ssum.at[i3b], xb.at[pl.ds(BH, BH)], sgb).wait()
        affine(xb)
        pltpu.async_copy(xb.at[pl.ds(BH, BH)], out_hbm.at[pl.ds(row3(i1), BH)], sob)

        @pl.when(t + 1 < PAIRS3)
        def _():
            pltpu.async_copy(idx_hbm.at[pl.ds(row3(i1 + 2), BH)], i3b, sib)
            pltpu.async_copy(x_hbm.at[pl.ds(row3(i1 + 2), BH)],
                             xb.at[pl.ds(0, BH)], sxb)
            pltpu.make_async_copy(idx_hbm.at[pl.ds(row3(i0 + 2), BH)], i3a, sia).wait()
            pltpu.make_async_copy(xa.at[pl.ds(BH, BH)],
                                  out_hbm.at[pl.ds(row3(i0), BH)], soa).wait()
            pltpu.async_copy(ssum.at[i3a], xa.at[pl.ds(BH, BH)], sga)

        return 0

    lax.fori_loop(0, PAIRS3, emit, 0)
    pltpu.make_async_copy(xa.at[pl.ds(BH, BH)],
                          out_hbm.at[pl.ds(row3(0), BH)], soa).wait()
    pltpu.make_async_copy(xb.at[pl.ds(BH, BH)],
                          out_hbm.at[pl.ds(row3(1), BH)], sob).wait()

    @pl.when(wid < EXTRA_W)
    def _():
        row = (NW * GPW + wid) * BH
        pltpu.sync_copy(idx_hbm.at[pl.ds(row, BH)], i3a)
        pltpu.sync_copy(x_hbm.at[pl.ds(row, BH)], xa.at[pl.ds(0, BH)])
        pltpu.sync_copy(ssum.at[i3a], xa.at[pl.ds(BH, BH)])
        affine(xa)
        pltpu.sync_copy(xa.at[pl.ds(BH, BH)], out_hbm.at[pl.ds(row, BH)])


_sc_kernel = pl.kernel(
    _body,
    out_type=jax.ShapeDtypeStruct((N, D), jnp.float32),
    mesh=_mesh,
    scratch_types=[
        pltpu.VMEM_SHARED((SPAD, D), jnp.float32),
        pltpu.VMEM_SHARED((SPAD,), jnp.float32),
        pltpu.VMEM((BL, D), jnp.float32),
        pltpu.VMEM((BL, D), jnp.float32),
        pltpu.VMEM((BL,), jnp.int32),
        pltpu.VMEM((BL,), jnp.int32),
        pltpu.VMEM((BL,), jnp.int32),
        pltpu.VMEM((BL,), jnp.int32),
        pltpu.VMEM((BH,), jnp.int32),
        pltpu.VMEM((BH,), jnp.int32),
        pltpu.VMEM((BL,), jnp.float32),
        pltpu.VMEM((D,), jnp.float32),
        pltpu.VMEM((D,), jnp.float32),
        pltpu.SemaphoreType.DMA,
        pltpu.SemaphoreType.DMA,
        pltpu.SemaphoreType.DMA,
        pltpu.SemaphoreType.DMA,
        pltpu.SemaphoreType.DMA,
        pltpu.SemaphoreType.DMA,
        pltpu.SemaphoreType.DMA,
        pltpu.SemaphoreType.DMA,
        pltpu.SemaphoreType.DMA,
        pltpu.SemaphoreType.DMA,
    ],
)


@jax.jit
def kernel(x, index, gamma, beta):
    idx = index.astype(jnp.int32)
    return _sc_kernel(x, idx,
                      gamma.reshape(D).astype(jnp.float32),
                      beta.reshape(D).astype(jnp.float32))
